# baseline (speedup 1.0000x reference)
import functools

import jax
import jax.numpy as jnp
from jax import lax
from jax.experimental import pallas as pl
from jax.experimental.pallas import tpu as pltpu
from jax.experimental.pallas import tpu_sc as plsc

F = 26
B = 16384
FIELD = 40000
LANES = 16
NC, NS = 2, 16
NW = NC * NS
BPW = B // NW
N = F * BPW

_mesh = plsc.VectorSubcoreMesh(core_axis_name="c", subcore_axis_name="s")


@functools.partial(
    pl.kernel,
    mesh=_mesh,
    out_type=jax.ShapeDtypeStruct((B,), jnp.float32),
    scratch_types=[
        pltpu.VMEM((N,), jnp.int32),
        pltpu.VMEM((N,), jnp.float32),
        pltpu.VMEM((BPW,), jnp.float32),
        pltpu.VMEM((LANES,), jnp.float32),
        pltpu.SemaphoreType.DMA,
    ],
)
def _emb_sum(xt_hbm, fc_hbm, bias_hbm, out_hbm, idx_v, vals_v, out_v, bias_v,
             sem):
    wid = lax.axis_index("s") * NC + lax.axis_index("c")
    base = wid * BPW

    copies = [
        pltpu.async_copy(
            xt_hbm.at[f, pl.ds(base, BPW)],
            idx_v.at[pl.ds(f * BPW, BPW)],
            sem,
        )
        for f in range(F)
    ]
    pltpu.sync_copy(bias_hbm, bias_v)
    for cp in copies:
        cp.wait()

    gathers = [
        pltpu.async_copy(
            fc_hbm.at[0].at[pl.ds(f * FIELD, FIELD)].at[
                idx_v.at[pl.ds(f * BPW, BPW)]],
            vals_v.at[pl.ds(f * BPW, BPW)],
            sem,
        )
        for f in range(F)
    ]
    for g in gathers:
        g.wait()

    def reduce_chunk(c, carry):
        acc = bias_v[...]
        for f in range(F):
            acc = acc + vals_v[pl.ds(f * BPW + c * LANES, LANES)]
        out_v[pl.ds(c * LANES, LANES)] = acc
        return carry

    lax.fori_loop(0, BPW // LANES, reduce_chunk, 0)

    pltpu.sync_copy(out_v, out_hbm.at[pl.ds(base, BPW)])


def kernel(x, fc_weight, bias):
    xt = x.T
    fc_t = fc_weight.T
    bias_b = jnp.broadcast_to(bias.astype(jnp.float32), (LANES,))
    out = _emb_sum(xt, fc_t, bias_b)
    return out.reshape(B, 1)
